# baseline (speedup 1.0000x reference)
import functools

import jax
import jax.numpy as jnp
from jax.experimental import pallas as pl
from jax.experimental.pallas import tpu as pltpu
from jax.experimental.pallas import tpu_sc as plsc

K = 10
NS = 16
CHUNK = 128
VOCAB_ = 400000
DIM_ = 300
HALF_ = VOCAB_ // 2
ROWS0_ = 1563 * CHUNK
ROWS1_ = VOCAB_ - ROWS0_


def _make_sc_body(row_start, nrows):
    nchunks = nrows // CHUNK

    def body(w_hbm, wv_hbm, num_hbm, sq_hbm, wbuf, buf, nbuf, sbuf, sems):
        wid = jax.lax.axis_index("s")
        nt = (nchunks - 1 - wid) // NS + 1

        pltpu.sync_copy(w_hbm, wbuf)
        lane = jax.lax.iota(jnp.int32, 16)
        m01 = jnp.where(lane >= 4, jnp.float32(1.0), jnp.float32(0.0))
        wjs = [wbuf[pl.ds(16 * j, 16)] for j in range(18)]
        wt = wbuf[pl.ds(284, 16)] * m01

        def copy_in(t, par):
            g = wid + NS * t
            return pltpu.make_async_copy(
                wv_hbm.at[pl.ds(row_start + g * CHUNK, CHUNK), :],
                buf.at[par], sems.at[par])

        copy_in(0, 0).start()

        def chunk_body(t, carry):
            par = jax.lax.rem(t, 2)
            g = wid + NS * t
            copy_in(t, par).wait()

            @pl.when(t + 1 < nt)
            def _():
                copy_in(t + 1, 1 - par).start()

            @plsc.parallel_loop(0, CHUNK, unroll=2)
            def _row(r):
                x = buf[par, r, pl.ds(0, 16)]
                acc_n = x * wjs[0]
                acc_s = x * x
                for j in range(1, 18):
                    x = buf[par, r, pl.ds(16 * j, 16)]
                    acc_n = acc_n + x * wjs[j]
                    acc_s = acc_s + x * x
                x = buf[par, r, pl.ds(284, 16)]
                acc_n = acc_n + x * wt
                xm = x * m01
                acc_s = acc_s + xm * xm
                pvec = jnp.full((16,), par, jnp.int32)
                rvec = jnp.full((16,), r, jnp.int32)
                lane0 = lane == 0
                plsc.store_scatter(nbuf, [pvec, rvec],
                                   jnp.full((16,), jnp.sum(acc_n), jnp.float32),
                                   mask=lane0)
                plsc.store_scatter(sbuf, [pvec, rvec],
                                   jnp.full((16,), jnp.sum(acc_s), jnp.float32),
                                   mask=lane0)

            pltpu.sync_copy(nbuf.at[par], num_hbm.at[pl.ds(g * CHUNK, CHUNK)])
            pltpu.sync_copy(sbuf.at[par], sq_hbm.at[pl.ds(g * CHUNK, CHUNK)])
            return carry

        jax.lax.fori_loop(0, nt, chunk_body, 0)

    return body


def _make_sc_half(row_start, nrows):
    return functools.partial(
        pl.kernel,
        out_type=(
            jax.ShapeDtypeStruct((nrows,), jnp.float32),
            jax.ShapeDtypeStruct((nrows,), jnp.float32),
        ),
        mesh=plsc.VectorSubcoreMesh(
            core_axis_name="c", subcore_axis_name="s", num_cores=1,
            num_subcores=NS),
        scratch_types=(
            pltpu.VMEM((DIM_,), jnp.float32),
            pltpu.VMEM((2, CHUNK, DIM_), jnp.float32),
            pltpu.VMEM((2, CHUNK), jnp.float32),
            pltpu.VMEM((2, CHUNK), jnp.float32),
            pltpu.SemaphoreType.DMA((2,)),
        ),
        compiler_params=pltpu.CompilerParams(needs_layout_passes=False),
    )(_make_sc_body(row_start, nrows))


_sc_half0 = _make_sc_half(0, ROWS0_)
_sc_half1 = _make_sc_half(ROWS0_, ROWS1_)


def _topk_kernel(w_ref, n_ref, s_ref, out_ref):
    wsq = jnp.sum(w_ref[...] * w_ref[...])
    s = n_ref[...] / (jnp.sqrt(s_ref[...] + 1e-9) * jnp.sqrt(wsq))
    rows = s.shape[0]
    row = jax.lax.broadcasted_iota(jnp.int32, (rows, 128), 0)
    col = jax.lax.broadcasted_iota(jnp.int32, (rows, 128), 1)
    flat = row * 128 + col
    big = jnp.int32(2147483647)
    for i in range(K):
        m = jnp.max(s)
        idx = jnp.min(jnp.where(s == m, flat, big))
        out_ref[i] = idx
        s = jnp.where(flat == idx, -jnp.inf, s)


def kernel(w, word2vec, k):
    vocab, dim = word2vec.shape
    n0, s0 = _sc_half0(w, word2vec)
    n1, s1 = _sc_half1(w, word2vec)
    num = jnp.concatenate([n0, n1])
    sq = jnp.concatenate([s0, s1])
    wcol = w.reshape(dim, 1)
    idx = pl.pallas_call(
        _topk_kernel,
        out_specs=pl.BlockSpec(memory_space=pltpu.SMEM),
        out_shape=jax.ShapeDtypeStruct((K,), jnp.int32),
    )(wcol, num.reshape(vocab // 128, 128), sq.reshape(vocab // 128, 128))
    return idx


# speedup vs baseline: 1.3867x; 1.3867x over previous
import functools

import jax
import jax.numpy as jnp
from jax.experimental import pallas as pl
from jax.experimental.pallas import tpu as pltpu
from jax.experimental.pallas import tpu_sc as plsc

K = 10
NC, NS = 2, 16
NW = NC * NS
CHUNK = 128
VOCAB_ = 400000
DIM_ = 300
BLOCK = 3200
F_TC = 147200
ROWS_SC = VOCAB_ - F_TC


def _sc_score_body(w_hbm, wv_hbm, num_hbm, sq_hbm, wbuf, buf, nbuf, sbuf, sems):
    nchunks = ROWS_SC // CHUNK
    wid = jax.lax.axis_index("s") * NC + jax.lax.axis_index("c")
    nt = (nchunks - 1 - wid) // NW + 1

    pltpu.sync_copy(w_hbm, wbuf)
    lane = jax.lax.iota(jnp.int32, 16)
    m01 = jnp.where(lane >= 4, jnp.float32(1.0), jnp.float32(0.0))
    wjs = [wbuf[pl.ds(16 * j, 16)] for j in range(18)]
    wt = wbuf[pl.ds(284, 16)] * m01

    def copy_in(t, par):
        g = wid + NW * t
        return pltpu.make_async_copy(
            wv_hbm.at[pl.ds(F_TC + g * CHUNK, CHUNK), :],
            buf.at[par], sems.at[par])

    copy_in(0, 0).start()

    def chunk_body(t, carry):
        par = jax.lax.rem(t, 2)
        g = wid + NW * t
        copy_in(t, par).wait()

        @pl.when(t + 1 < nt)
        def _():
            copy_in(t + 1, 1 - par).start()

        @plsc.parallel_loop(0, CHUNK, unroll=2)
        def _row(r):
            x = buf[par, r, pl.ds(0, 16)]
            acc_n = x * wjs[0]
            acc_s = x * x
            for j in range(1, 18):
                x = buf[par, r, pl.ds(16 * j, 16)]
                acc_n = acc_n + x * wjs[j]
                acc_s = acc_s + x * x
            x = buf[par, r, pl.ds(284, 16)]
            acc_n = acc_n + x * wt
            xm = x * m01
            acc_s = acc_s + xm * xm
            pvec = jnp.full((16,), par, jnp.int32)
            rvec = jnp.full((16,), r, jnp.int32)
            lane0 = lane == 0
            plsc.store_scatter(nbuf, [pvec, rvec],
                               jnp.full((16,), jnp.sum(acc_n), jnp.float32),
                               mask=lane0)
            plsc.store_scatter(sbuf, [pvec, rvec],
                               jnp.full((16,), jnp.sum(acc_s), jnp.float32),
                               mask=lane0)

        pltpu.sync_copy(nbuf.at[par], num_hbm.at[pl.ds(g * CHUNK, CHUNK)])
        pltpu.sync_copy(sbuf.at[par], sq_hbm.at[pl.ds(g * CHUNK, CHUNK)])
        return carry

    jax.lax.fori_loop(0, nt, chunk_body, 0)


_sc_score = functools.partial(
    pl.kernel,
    out_type=(
        jax.ShapeDtypeStruct((ROWS_SC,), jnp.float32),
        jax.ShapeDtypeStruct((ROWS_SC,), jnp.float32),
    ),
    mesh=plsc.VectorSubcoreMesh(
        core_axis_name="c", subcore_axis_name="s", num_cores=NC,
        num_subcores=NS),
    scratch_types=(
        pltpu.VMEM((DIM_,), jnp.float32),
        pltpu.VMEM((2, CHUNK, DIM_), jnp.float32),
        pltpu.VMEM((2, CHUNK), jnp.float32),
        pltpu.VMEM((2, CHUNK), jnp.float32),
        pltpu.SemaphoreType.DMA((2,)),
    ),
    compiler_params=pltpu.CompilerParams(needs_layout_passes=False),
)(_sc_score_body)


def _tc_score_kernel(w_ref, wv_ref, num_ref, sq_ref):
    tile = wv_ref[...]
    wcol = w_ref[...]
    num = jnp.dot(tile, wcol, preferred_element_type=jnp.float32)
    sq = jnp.dot(tile * tile, jnp.ones_like(wcol),
                 preferred_element_type=jnp.float32)
    num_ref[...] = num.reshape(1, 1, -1)
    sq_ref[...] = sq.reshape(1, 1, -1)


def _topk_kernel(w_ref, n_ref, s_ref, out_ref):
    wsq = jnp.sum(w_ref[...] * w_ref[...])
    s = n_ref[...] / (jnp.sqrt(s_ref[...] + 1e-9) * jnp.sqrt(wsq))
    rows = s.shape[0]
    row = jax.lax.broadcasted_iota(jnp.int32, (rows, 128), 0)
    col = jax.lax.broadcasted_iota(jnp.int32, (rows, 128), 1)
    flat = row * 128 + col
    big = jnp.int32(2147483647)
    for i in range(K):
        m = jnp.max(s)
        idx = jnp.min(jnp.where(s == m, flat, big))
        out_ref[i] = idx
        s = jnp.where(flat == idx, -jnp.inf, s)


def kernel(w, word2vec, k):
    vocab, dim = word2vec.shape
    wcol = w.reshape(dim, 1)

    sc_n, sc_sq = _sc_score(w, word2vec)

    nb = F_TC // BLOCK
    tc_n, tc_sq = pl.pallas_call(
        _tc_score_kernel,
        grid=(nb,),
        in_specs=[
            pl.BlockSpec((dim, 1), lambda i: (0, 0)),
            pl.BlockSpec((BLOCK, dim), lambda i: (i, 0)),
        ],
        out_specs=[
            pl.BlockSpec((1, 1, BLOCK), lambda i: (i, 0, 0)),
            pl.BlockSpec((1, 1, BLOCK), lambda i: (i, 0, 0)),
        ],
        out_shape=[
            jax.ShapeDtypeStruct((nb, 1, BLOCK), jnp.float32),
            jax.ShapeDtypeStruct((nb, 1, BLOCK), jnp.float32),
        ],
    )(wcol, word2vec)

    num = jnp.concatenate([tc_n.reshape(-1), sc_n])
    sq = jnp.concatenate([tc_sq.reshape(-1), sc_sq])
    idx = pl.pallas_call(
        _topk_kernel,
        out_specs=pl.BlockSpec(memory_space=pltpu.SMEM),
        out_shape=jax.ShapeDtypeStruct((K,), jnp.int32),
    )(wcol, num.reshape(vocab // 128, 128), sq.reshape(vocab // 128, 128))
    return idx
